# baseline (speedup 1.0000x reference)
import jax
import jax.numpy as jnp
import numpy as np
from jax import lax
from jax.experimental import pallas as pl
from jax.experimental.pallas import tpu as pltpu
from jax.experimental.pallas import tpu_sc as plsc

BATCH = 16384
EMBED = 64
VOCAB = 1000000
NUM_CORES = 2
NUM_SUBCORES = 16
LANES = 16
NUM_WORKERS = NUM_CORES * NUM_SUBCORES
CHUNK = BATCH // NUM_WORKERS
GROUPS = CHUNK // LANES

BLK_V = 131072
N_VBLK = 8
N_DBLK = EMBED // 8
POOL = N_DBLK * N_VBLK * 8 * BLK_V

HALF = EMBED // 2
HWORDS = HALF * CHUNK
TWORDS = EMBED * CHUNK


def _detile_body(in_ref, o_ref):
    o_ref[...] = in_ref[...].reshape(8 * BLK_V)


@jax.jit
def _detile(t):
    return pl.pallas_call(
        _detile_body,
        grid=(N_DBLK, N_VBLK),
        in_specs=[pl.BlockSpec((8, BLK_V), lambda a, c: (a, c))],
        out_specs=pl.BlockSpec((8 * BLK_V,), lambda a, c: (a * N_VBLK + c,)),
        out_shape=jax.ShapeDtypeStruct((POOL,), jnp.float32),
    )(t)


def _ugather_body(gidxu_hbm, uflat_hbm, uvals_hbm, idxu, idxu2, vu, vu2, sem):
    wid = lax.axis_index("s") * NUM_CORES + lax.axis_index("c")
    slab = wid * TWORDS

    pltpu.sync_copy(gidxu_hbm.at[pl.ds(slab, HWORDS)], idxu)
    pltpu.sync_copy(gidxu_hbm.at[pl.ds(slab + HWORDS, HWORDS)], idxu2)
    c0 = pltpu.async_copy(uflat_hbm.at[idxu], vu, sem)
    c1 = pltpu.async_copy(uflat_hbm.at[idxu2], vu2, sem)
    c0.wait()
    pltpu.sync_copy(vu, uvals_hbm.at[pl.ds(slab, HWORDS)])
    c1.wait()
    pltpu.sync_copy(vu2, uvals_hbm.at[pl.ds(slab + HWORDS, HWORDS)])


@jax.jit
def _ugather(gidxu, uflat):
    mesh = plsc.VectorSubcoreMesh(core_axis_name="c", subcore_axis_name="s")
    cp = pltpu.CompilerParams(
        needs_layout_passes=False,
        use_tc_tiling_on_sc=False,
    )
    run = pl.kernel(
        _ugather_body,
        out_type=jax.ShapeDtypeStruct((EMBED * BATCH,), jnp.float32),
        mesh=mesh,
        scratch_types=[
            pltpu.VMEM((HWORDS,), jnp.int32),
            pltpu.VMEM((HWORDS,), jnp.int32),
            pltpu.VMEM((HWORDS,), jnp.float32),
            pltpu.VMEM((HWORDS,), jnp.float32),
            pltpu.SemaphoreType.DMA,
        ],
        compiler_params=cp,
    )
    return run(gidxu, uflat)


def _bpr_body(gidxp_hbm, gidxn_hbm, uvals_hbm, iflat_hbm, out_hbm,
              idxp, idxn, idxp2, idxn2, vu, vp, vn, outv, sem):
    wid = lax.axis_index("s") * NUM_CORES + lax.axis_index("c")
    base = wid * CHUNK
    slab = wid * TWORDS

    pltpu.sync_copy(gidxp_hbm.at[pl.ds(slab, HWORDS)], idxp)
    pltpu.sync_copy(gidxn_hbm.at[pl.ds(slab, HWORDS)], idxn)
    pltpu.sync_copy(gidxp_hbm.at[pl.ds(slab + HWORDS, HWORDS)], idxp2)
    pltpu.sync_copy(gidxn_hbm.at[pl.ds(slab + HWORDS, HWORDS)], idxn2)

    for h in range(2):
        off = slab + h * HWORDS
        ip = idxp if h == 0 else idxp2
        inn = idxn if h == 0 else idxn2
        cu = pltpu.async_copy(uvals_hbm.at[pl.ds(off, HWORDS)], vu, sem)
        cp_ = pltpu.async_copy(iflat_hbm.at[ip], vp, sem)
        cn = pltpu.async_copy(iflat_hbm.at[inn], vn, sem)
        cu.wait()
        cp_.wait()
        cn.wait()

        if h == 0:
            @pl.loop(0, GROUPS)
            def _dot_a(g):
                sl = pl.ds(g * LANES, LANES)
                acc = jnp.zeros((LANES,), jnp.float32)
                for dd in range(HALF):
                    vsl = pl.ds(dd * CHUNK + g * LANES, LANES)
                    acc = acc + vu[vsl] * (vp[vsl] - vn[vsl])
                outv[sl] = acc
        else:
            @pl.loop(0, GROUPS)
            def _dot_b(g):
                sl = pl.ds(g * LANES, LANES)
                acc = outv[sl]
                for dd in range(HALF):
                    vsl = pl.ds(dd * CHUNK + g * LANES, LANES)
                    acc = acc + vu[vsl] * (vp[vsl] - vn[vsl])
                outv[sl] = acc

    pltpu.sync_copy(outv, out_hbm.at[pl.ds(base, CHUNK)])


@jax.jit
def _bpr_sc(gidxp, gidxn, uvals, iflat):
    mesh = plsc.VectorSubcoreMesh(core_axis_name="c", subcore_axis_name="s")
    cp = pltpu.CompilerParams(
        needs_layout_passes=False,
        use_tc_tiling_on_sc=False,
    )
    run = pl.kernel(
        _bpr_body,
        out_type=jax.ShapeDtypeStruct((BATCH,), jnp.float32),
        mesh=mesh,
        scratch_types=[
            pltpu.VMEM((HWORDS,), jnp.int32),
            pltpu.VMEM((HWORDS,), jnp.int32),
            pltpu.VMEM((HWORDS,), jnp.int32),
            pltpu.VMEM((HWORDS,), jnp.int32),
            pltpu.VMEM((HWORDS,), jnp.float32),
            pltpu.VMEM((HWORDS,), jnp.float32),
            pltpu.VMEM((HWORDS,), jnp.float32),
            pltpu.VMEM((CHUNK,), jnp.float32),
            pltpu.SemaphoreType.DMA,
        ],
        compiler_params=cp,
    )
    return run(gidxp, gidxn, uvals, iflat)


def _pool_indices(v):
    b = (v >> 17) * 1048576 + (v & 131071)
    d = jnp.arange(EMBED, dtype=jnp.int32)
    c = (d >> 3) * 8388608 + (d & 7) * 131072
    arr = b.reshape(NUM_WORKERS, 1, CHUNK) + c.reshape(1, EMBED, 1)
    return arr.reshape(-1)


def kernel(batch, user_memory, item_memory):
    gidxu = _pool_indices(batch[:, 0])
    gidxp = _pool_indices(batch[:, 1])
    gidxn = _pool_indices(batch[:, 2])
    uflat = _detile(user_memory.T)
    uvals = _ugather(gidxu, uflat)
    iflat = _detile(item_memory.T)
    return _bpr_sc(gidxp, gidxn, uvals, iflat)
